# baseline (speedup 1.0000x reference)
import functools

import jax
import jax.numpy as jnp
from jax import lax
from jax.experimental import pallas as pl
from jax.experimental.pallas import tpu as pltpu
from jax.experimental.pallas import tpu_sc as plsc

N_COINS = 100000
EMBED_DIM = 64
BATCH = 16384
HIST = 50
HIST_P = 56
DIM_P = 128

NC, NS = 2, 16
NW = NC * NS
BATCH_PER_W = BATCH // NW
NB = 16
CHUNKS_PER_W = BATCH_PER_W // NB
NBUF = 2


def _gather_kernel(table_hbm, idx_hbm, out_hbm,
                   idx_v, rows0, rows1, gsem0, gsem1, wsem0, wsem1):
    wid = lax.axis_index("s") * NC + lax.axis_index("c")
    batch0 = wid * BATCH_PER_W
    rows = (rows0, rows1)
    gsem = (gsem0, gsem1)
    wsem = (wsem0, wsem1)

    pltpu.sync_copy(idx_hbm.at[pl.ds(batch0, BATCH_PER_W)], idx_v)

    def fire_gather(c, s):
        for b in range(NB):
            pltpu.async_copy(
                table_hbm.at[idx_v.at[c * NB + b]],
                rows[s].at[b],
                gsem[s])

    def drain_gather(s):
        pltpu.make_async_copy(
            out_hbm.at[pl.ds(0, NB), pl.ds(0, HIST), pl.ds(0, EMBED_DIM)],
            rows[s],
            gsem[s]).wait()

    def fire_write(c, s):
        pltpu.async_copy(
            rows[s],
            out_hbm.at[pl.ds(batch0 + c * NB, NB), pl.ds(0, HIST),
                       pl.ds(0, EMBED_DIM)],
            wsem[s])

    def drain_write(s):
        pltpu.make_async_copy(
            rows[s],
            out_hbm.at[pl.ds(batch0, NB), pl.ds(0, HIST),
                       pl.ds(0, EMBED_DIM)],
            wsem[s]).wait()

    for s in range(NBUF):
        fire_gather(s, s)

    def body(i, carry):
        g = i * NBUF
        for s in range(NBUF):
            c = g + s
            drain_gather(s)
            fire_write(c, s)
            drain_write(s)
            fire_gather(c + NBUF, s)
        return carry

    lax.fori_loop(0, (CHUNKS_PER_W - NBUF) // NBUF, body, 0)

    for s in range(NBUF):
        drain_gather(s)
        fire_write(CHUNKS_PER_W - NBUF + s, s)
        drain_write(s)


@functools.cache
def _build():
    return pl.kernel(
        _gather_kernel,
        out_type=jax.ShapeDtypeStruct((BATCH, HIST_P, DIM_P), jnp.float32),
        mesh=plsc.VectorSubcoreMesh(
            core_axis_name="c", subcore_axis_name="s",
            num_cores=NC, num_subcores=NS,
        ),
        scratch_types=[
            pltpu.VMEM((BATCH_PER_W, HIST), jnp.int32),
            pltpu.VMEM((NB, HIST, EMBED_DIM), jnp.float32),
            pltpu.VMEM((NB, HIST, EMBED_DIM), jnp.float32),
            pltpu.SemaphoreType.DMA,
            pltpu.SemaphoreType.DMA,
            pltpu.SemaphoreType.DMA,
            pltpu.SemaphoreType.DMA,
        ],
        compiler_params=pltpu.CompilerParams(use_tc_tiling_on_sc=False),
    )


def kernel(coin_id, table):
    out_p = _build()(table, coin_id.astype(jnp.int32))
    return out_p[:, :HIST, :EMBED_DIM]


# speedup vs baseline: 1.0001x; 1.0001x over previous
import functools

import jax
import jax.numpy as jnp
from jax import lax
from jax.experimental import pallas as pl
from jax.experimental.pallas import tpu as pltpu
from jax.experimental.pallas import tpu_sc as plsc

N_COINS = 100000
EMBED_DIM = 64
BATCH = 16384
HIST = 50
HIST_P = 56
DIM_P = 128

NC, NS = 2, 16
NW = NC * NS
BATCH_PER_W = BATCH // NW
NB = 8
CHUNKS_PER_W = BATCH_PER_W // NB
NBUF = 2


def _gather_kernel(table_hbm, idx_hbm, out_hbm,
                   idx_v, rows0, rows1, gsem0, gsem1, wsem0, wsem1):
    wid = lax.axis_index("s") * NC + lax.axis_index("c")
    batch0 = wid * BATCH_PER_W
    rows = (rows0, rows1)
    gsem = (gsem0, gsem1)
    wsem = (wsem0, wsem1)

    pltpu.sync_copy(idx_hbm.at[pl.ds(batch0, BATCH_PER_W)], idx_v)

    def fire_gather(c, s):
        for b in range(NB):
            pltpu.async_copy(
                table_hbm.at[idx_v.at[c * NB + b]],
                rows[s].at[b],
                gsem[s])

    def drain_gather(s):
        pltpu.make_async_copy(
            out_hbm.at[pl.ds(0, NB), pl.ds(0, HIST), pl.ds(0, EMBED_DIM)],
            rows[s],
            gsem[s]).wait()

    def fire_write(c, s):
        pltpu.async_copy(
            rows[s],
            out_hbm.at[pl.ds(batch0 + c * NB, NB), pl.ds(0, HIST),
                       pl.ds(0, EMBED_DIM)],
            wsem[s])

    def drain_write(s):
        pltpu.make_async_copy(
            rows[s],
            out_hbm.at[pl.ds(batch0, NB), pl.ds(0, HIST),
                       pl.ds(0, EMBED_DIM)],
            wsem[s]).wait()

    for s in range(NBUF):
        fire_gather(s, s)

    def body(i, carry):
        g = i * NBUF
        for s in range(NBUF):
            c = g + s
            drain_gather(s)
            fire_write(c, s)
            drain_write(s)
            fire_gather(c + NBUF, s)
        return carry

    lax.fori_loop(0, (CHUNKS_PER_W - NBUF) // NBUF, body, 0)

    for s in range(NBUF):
        drain_gather(s)
        fire_write(CHUNKS_PER_W - NBUF + s, s)
        drain_write(s)


@functools.cache
def _build():
    return pl.kernel(
        _gather_kernel,
        out_type=jax.ShapeDtypeStruct((BATCH, HIST_P, DIM_P), jnp.float32),
        mesh=plsc.VectorSubcoreMesh(
            core_axis_name="c", subcore_axis_name="s",
            num_cores=NC, num_subcores=NS,
        ),
        scratch_types=[
            pltpu.VMEM((BATCH_PER_W, HIST), jnp.int32),
            pltpu.VMEM((NB, HIST, EMBED_DIM), jnp.float32),
            pltpu.VMEM((NB, HIST, EMBED_DIM), jnp.float32),
            pltpu.SemaphoreType.DMA,
            pltpu.SemaphoreType.DMA,
            pltpu.SemaphoreType.DMA,
            pltpu.SemaphoreType.DMA,
        ],
        compiler_params=pltpu.CompilerParams(use_tc_tiling_on_sc=False),
    )


def kernel(coin_id, table):
    out_p = _build()(table, coin_id.astype(jnp.int32))
    return out_p[:, :HIST, :EMBED_DIM]
